# baseline (speedup 1.0000x reference)
import functools

import jax
import jax.numpy as jnp
from jax import lax
from jax.experimental import pallas as pl
from jax.experimental.pallas import tpu as pltpu
from jax.experimental.pallas import tpu_sc as plsc

C = 40
D = 256
N = 50000
INV_T = 10.0
EPS = 1e-10

NC = 2
NS = 16
W = NC * NS
CH = 80
NCHUNK = N // CH
ITERS = -(-NCHUNK // W)


def _sc_body(emb, lab, out, rows_v, lab_v, acc_v, sem_r, sem_l):
    cid = lax.axis_index("c")
    sid = lax.axis_index("s")
    wid = sid * NC + cid

    zero16 = jnp.zeros((16,), jnp.float32)

    def _zero_row(r, carry):
        for j in range(D // 16):
            acc_v[r, pl.ds(j * 16, 16)] = zero16
        return carry

    lax.fori_loop(0, C, _zero_row, 0)

    def _start(it, b):
        c = wid + W * it

        @pl.when(c < NCHUNK)
        def _():
            base = pl.multiple_of(c * CH, 8)
            pltpu.make_async_copy(
                lab.at[pl.ds(base, CH)], lab_v.at[b], sem_l.at[b]
            ).start()
            pltpu.make_async_copy(
                emb.at[pl.ds(base, CH)], rows_v.at[b], sem_r.at[b]
            ).start()

    def _compute(it, b):
        c = wid + W * it

        @pl.when(c < NCHUNK)
        def _():
            pltpu.make_async_copy(
                lab.at[pl.ds(0, CH)], lab_v.at[b], sem_l.at[b]
            ).wait()
            pltpu.make_async_copy(
                emb.at[pl.ds(0, CH)], rows_v.at[b], sem_r.at[b]
            ).wait()

            @plsc.parallel_loop(0, CH // 16, unroll=2)
            def _group(g):
                labv = lab_v[b, pl.ds(g * 16, 16)]
                for k in range(16):
                    l = labv[k]
                    i = g * 16 + k
                    for j in range(D // 16):
                        sl = pl.ds(j * 16, 16)
                        plsc.addupdate(acc_v.at[l, sl], rows_v[b, i, sl])

    _start(0, 0)

    def _pair(it2, carry):
        it_a = 2 * it2
        _start(it_a + 1, 1)
        _compute(it_a, 0)
        _start(it_a + 2, 0)
        _compute(it_a + 1, 1)
        return carry

    lax.fori_loop(0, ITERS // 2, _pair, 0)

    pltpu.sync_copy(acc_v, out.at[wid])


@functools.cache
def _sc_segsum():
    return pl.kernel(
        _sc_body,
        out_type=jax.ShapeDtypeStruct((W, C, D), jnp.float32),
        mesh=plsc.VectorSubcoreMesh(core_axis_name="c", subcore_axis_name="s"),
        scratch_types=[
            pltpu.VMEM((2, CH, D), jnp.float32),
            pltpu.VMEM((2, CH), jnp.int32),
            pltpu.VMEM((C, D), jnp.float32),
            pltpu.SemaphoreType.DMA((2,)),
            pltpu.SemaphoreType.DMA((2,)),
        ],
    )


B = 10000
GRID = N // B


def _tc_body(part_ref, emb_ref, out_ref, pn_ref):
    @pl.when(pl.program_id(0) == 0)
    def _():
        s = jnp.sum(part_ref[...], axis=0)
        nrm = jnp.sqrt(jnp.sum(s * s, axis=1, keepdims=True))
        pn_ref[...] = s / jnp.maximum(nrm, EPS)

    e = emb_ref[...]
    g = lax.dot_general(
        e, pn_ref[...], (((1,), (1,)), ((), ())),
        preferred_element_type=jnp.float32,
    )
    en = jnp.sqrt(jnp.sum(e * e, axis=1, keepdims=True))
    out_ref[...] = g * (INV_T / jnp.maximum(en, EPS))


_tc_similarity = pl.pallas_call(
    _tc_body,
    grid=(GRID,),
    in_specs=[
        pl.BlockSpec((W, C, D), lambda i: (0, 0, 0)),
        pl.BlockSpec((B, D), lambda i: (i, 0)),
    ],
    out_specs=pl.BlockSpec((B, C), lambda i: (i, 0)),
    out_shape=jax.ShapeDtypeStruct((N, C), jnp.float32),
    scratch_shapes=[pltpu.VMEM((C, D), jnp.float32)],
)


def kernel(embed, labels):
    lab32 = labels.astype(jnp.int32)
    partials = _sc_segsum()(embed, lab32)
    return _tc_similarity(partials, embed)


# speedup vs baseline: 1.4669x; 1.4669x over previous
import functools

import jax
import jax.numpy as jnp
from jax import lax
from jax.experimental import pallas as pl
from jax.experimental.pallas import tpu as pltpu
from jax.experimental.pallas import tpu_sc as plsc

C = 40
D = 256
N = 50000
INV_T = 10.0
EPS = 1e-10

NC = 2
NS = 16
W = NC * NS
CH = 80
NCHUNK = N // CH
ITERS = -(-NCHUNK // W)


def _sc_body(emb, lab, out, rows_v, lab_v, acc_v, sem_r, sem_l):
    cid = lax.axis_index("c")
    sid = lax.axis_index("s")
    wid = sid * NC + cid

    zero16 = jnp.zeros((16,), jnp.float32)

    def _zero_row(r, carry):
        for j in range(D // 16):
            acc_v[r, pl.ds(j * 16, 16)] = zero16
        return carry

    lax.fori_loop(0, C, _zero_row, 0)

    def _start(it, b):
        c = wid + W * it

        @pl.when(c < NCHUNK)
        def _():
            base = pl.multiple_of(c * CH, 8)
            pltpu.make_async_copy(
                lab.at[pl.ds(base, CH)], lab_v.at[b], sem_l.at[b]
            ).start()
            pltpu.make_async_copy(
                emb.at[pl.ds(base, CH)], rows_v.at[b], sem_r.at[b]
            ).start()

    def _compute(it, b):
        c = wid + W * it

        @pl.when(c < NCHUNK)
        def _():
            pltpu.make_async_copy(
                lab.at[pl.ds(0, CH)], lab_v.at[b], sem_l.at[b]
            ).wait()
            pltpu.make_async_copy(
                emb.at[pl.ds(0, CH)], rows_v.at[b], sem_r.at[b]
            ).wait()

            @plsc.parallel_loop(0, CH // 16)
            def _group(g):
                labv = lab_v[b, pl.ds(g * 16, 16)]
                ls = [labv[k] for k in range(16)]
                for k in range(16):
                    i = g * 16 + k
                    vals = [rows_v[b, i, pl.ds(j * 16, 16)] for j in range(D // 16)]
                    for j in range(D // 16):
                        plsc.addupdate(acc_v.at[ls[k], pl.ds(j * 16, 16)], vals[j])

    _start(0, 0)

    def _pair(it2, carry):
        it_a = 2 * it2
        _start(it_a + 1, 1)
        _compute(it_a, 0)
        _start(it_a + 2, 0)
        _compute(it_a + 1, 1)
        return carry

    lax.fori_loop(0, ITERS // 2, _pair, 0)

    pltpu.sync_copy(acc_v, out.at[wid])


@functools.cache
def _sc_segsum():
    return pl.kernel(
        _sc_body,
        out_type=jax.ShapeDtypeStruct((W, C, D), jnp.float32),
        mesh=plsc.VectorSubcoreMesh(core_axis_name="c", subcore_axis_name="s"),
        scratch_types=[
            pltpu.VMEM((2, CH, D), jnp.float32),
            pltpu.VMEM((2, CH), jnp.int32),
            pltpu.VMEM((C, D), jnp.float32),
            pltpu.SemaphoreType.DMA((2,)),
            pltpu.SemaphoreType.DMA((2,)),
        ],
    )


B = 5000
GRID = N // B


def _tc_body(part_ref, emb_ref, out_ref, pn_ref):
    @pl.when(pl.program_id(0) == 0)
    def _():
        s = jnp.sum(part_ref[...], axis=0)
        nrm = jnp.sqrt(jnp.sum(s * s, axis=1, keepdims=True))
        pn_ref[...] = s / jnp.maximum(nrm, EPS)

    e = emb_ref[...]
    g = lax.dot_general(
        e, pn_ref[...], (((1,), (1,)), ((), ())),
        preferred_element_type=jnp.float32,
    )
    en = jnp.sqrt(jnp.sum(e * e, axis=1, keepdims=True))
    out_ref[...] = g * (INV_T / jnp.maximum(en, EPS))


_tc_similarity = pl.pallas_call(
    _tc_body,
    grid=(GRID,),
    in_specs=[
        pl.BlockSpec((W, C, D), lambda i: (0, 0, 0)),
        pl.BlockSpec((B, D), lambda i: (i, 0)),
    ],
    out_specs=pl.BlockSpec((B, C), lambda i: (i, 0)),
    out_shape=jax.ShapeDtypeStruct((N, C), jnp.float32),
    scratch_shapes=[pltpu.VMEM((C, D), jnp.float32)],
)


def kernel(embed, labels):
    lab32 = labels.astype(jnp.int32)
    partials = _sc_segsum()(embed, lab32)
    return _tc_similarity(partials, embed)
